# baseline (speedup 1.0000x reference)
import functools

import jax
import jax.numpy as jnp
from jax import lax
from jax.experimental import pallas as pl
from jax.experimental.pallas import tpu as pltpu
from jax.experimental.pallas import tpu_sc as plsc

_NC = 2
_NS = 16
_NW = _NC * _NS

_D = 64
_SPLIT = 128


def _make_lookup(S, T):
    assert S % (2 * _NW) == 0
    s_per_w = S // _NW
    t2 = T - _SPLIT
    assert 0 < t2 <= 128 and _SPLIT % 8 == 0 and t2 % 8 == 0
    mesh = plsc.VectorSubcoreMesh(core_axis_name="c", subcore_axis_name="s")

    @functools.partial(
        pl.kernel,
        out_type=jax.ShapeDtypeStruct((S, T, _D), jnp.float32),
        mesh=mesh,
        scratch_types=[
            pltpu.VMEM((s_per_w, T), jnp.int32),
            pltpu.VMEM((T, _D), jnp.float32),
            pltpu.VMEM((T, _D), jnp.float32),
            pltpu.SemaphoreType.DMA,
            pltpu.SemaphoreType.DMA,
        ],
        compiler_params=pltpu.CompilerParams(use_tc_tiling_on_sc=False),
    )
    def lookup(table_hbm, idx_hbm, out_hbm, idx_v, rows0, rows1, sem0, sem1):
        wid = lax.axis_index("s") * _NC + lax.axis_index("c")
        s_base = wid * s_per_w
        pltpu.sync_copy(idx_hbm.at[wid], idx_v)

        bufs = (rows0, rows1)
        sems = (sem0, sem1)

        def fire(s, b):
            pltpu.async_copy(
                table_hbm.at[idx_v.at[s, pl.ds(0, _SPLIT)]],
                bufs[b].at[pl.ds(0, _SPLIT)],
                sems[b],
            )
            pltpu.async_copy(
                table_hbm.at[idx_v.at[s, pl.ds(_SPLIT, t2)]],
                bufs[b].at[pl.ds(_SPLIT, t2)],
                sems[b],
            )

        def drain(b):
            pltpu.make_async_copy(
                table_hbm.at[idx_v.at[0, pl.ds(0, _SPLIT)]],
                bufs[b].at[pl.ds(0, _SPLIT)],
                sems[b],
            ).wait()
            pltpu.make_async_copy(
                table_hbm.at[idx_v.at[0, pl.ds(_SPLIT, t2)]],
                bufs[b].at[pl.ds(_SPLIT, t2)],
                sems[b],
            ).wait()

        def copy_out(s, b):
            pltpu.sync_copy(bufs[b], out_hbm.at[s_base + s])

        fire(0, 0)

        def body(i0, _):
            s0 = 2 * i0
            fire(s0 + 1, 1)
            drain(0)
            copy_out(s0, 0)
            fire(s0 + 2, 0)
            drain(1)
            copy_out(s0 + 1, 1)
            return 0

        lax.fori_loop(0, (s_per_w - 2) // 2, body, 0)

        fire(s_per_w - 1, 1)
        drain(0)
        copy_out(s_per_w - 2, 0)
        drain(1)
        copy_out(s_per_w - 1, 1)

    return lookup


def kernel(sen_ids, table):
    S, T = sen_ids.shape
    idx3 = sen_ids.astype(jnp.int32).reshape(_NW, S // _NW, T)
    return _make_lookup(S, T)(table, idx3)


# speedup vs baseline: 1.3191x; 1.3191x over previous
import functools

import jax
import jax.numpy as jnp
from jax import lax
from jax.experimental import pallas as pl
from jax.experimental.pallas import tpu as pltpu
from jax.experimental.pallas import tpu_sc as plsc

_NC = 2
_NS = 16
_NW = _NC * _NS

_D = 64
_DP = 128
_CHUNK = 128
_G = 2


def _make_lookup(B):
    rows_per_group = _G * _CHUNK
    assert B % (_NW * rows_per_group) == 0
    per_w = B // _NW
    ngroups = per_w // rows_per_group
    assert ngroups % 2 == 0 and ngroups >= 4
    mesh = plsc.VectorSubcoreMesh(core_axis_name="c", subcore_axis_name="s")

    @functools.partial(
        pl.kernel,
        out_type=jax.ShapeDtypeStruct((B, _DP), jnp.float32),
        mesh=mesh,
        scratch_types=[
            pltpu.VMEM((per_w // _CHUNK, _CHUNK), jnp.int32),
            pltpu.VMEM((rows_per_group, _DP), jnp.float32),
            pltpu.VMEM((rows_per_group, _DP), jnp.float32),
            pltpu.SemaphoreType.DMA,
            pltpu.SemaphoreType.DMA,
        ],
    )
    def lookup(table_hbm, idx_hbm, out_hbm, idx_v, rows0, rows1, sem0, sem1):
        wid = lax.axis_index("s") * _NC + lax.axis_index("c")
        base = pl.multiple_of(wid * per_w, _CHUNK)
        pltpu.sync_copy(idx_hbm.at[wid], idx_v)

        bufs = (rows0, rows1)
        sems = (sem0, sem1)

        def fire(t, b):
            for i in range(_G):
                pltpu.async_copy(
                    table_hbm.at[idx_v.at[t * _G + i]],
                    bufs[b].at[pl.ds(i * _CHUNK, _CHUNK)],
                    sems[b],
                )

        def drain(b):
            for i in range(_G):
                pltpu.make_async_copy(
                    table_hbm.at[idx_v.at[i]],
                    bufs[b].at[pl.ds(i * _CHUNK, _CHUNK)],
                    sems[b],
                ).wait()

        def copy_out(t, b):
            off = pl.multiple_of(base + t * rows_per_group, _CHUNK)
            pltpu.sync_copy(bufs[b], out_hbm.at[pl.ds(off, rows_per_group)])

        fire(0, 0)

        def body(i0, _):
            t0 = 2 * i0
            fire(t0 + 1, 1)
            drain(0)
            copy_out(t0, 0)
            fire(t0 + 2, 0)
            drain(1)
            copy_out(t0 + 1, 1)
            return 0

        lax.fori_loop(0, (ngroups - 2) // 2, body, 0)

        fire(ngroups - 1, 1)
        drain(0)
        copy_out(ngroups - 2, 0)
        drain(1)
        copy_out(ngroups - 1, 1)

    return lookup


def kernel(sen_ids, table):
    S, T = sen_ids.shape
    B = S * T
    table_p = lax.pad(table, jnp.float32(0), ((0, 0, 0), (0, _DP - _D, 0)))
    idx = sen_ids.reshape(-1).astype(jnp.int32)
    idx3 = idx.reshape(_NW, B // (_NW * _CHUNK), _CHUNK)
    out = _make_lookup(B)(table_p, idx3)
    return out.reshape(S, T, _DP)[..., :_D]
